# baseline (speedup 1.0000x reference)
import functools
import math

import jax
import jax.numpy as jnp
from jax import lax
from jax.experimental import pallas as pl
from jax.experimental.pallas import tpu as pltpu
from jax.experimental.pallas import tpu_sc as plsc

ALPHA = 0.1
THETA = 0.5
BETA = math.log(THETA / 2 + 1.0)

NC = 2
NS = 16
NW = NC * NS
L = 16
C = 96
BK = 9


def _sc_spmm_kernel(N, D, nblk):
    rpw = N // NS

    mesh = plsc.VectorSubcoreMesh(core_axis_name="c", subcore_axis_name="s")

    @functools.partial(
        pl.kernel,
        out_type=jax.ShapeDtypeStruct((NC, N, D), jnp.float32),
        mesh=mesh,
        scratch_types=[
            pltpu.VMEM((2, BK, C), jnp.int32),
            pltpu.VMEM((2, BK, C), jnp.int32),
            pltpu.VMEM((2, BK, C), jnp.float32),
            pltpu.VMEM((3, C, D), jnp.float32),
            pltpu.VMEM_SHARED((N, D), jnp.float32),
            pltpu.SemaphoreType.DMA,
            pltpu.SemaphoreType.DMA,
            pltpu.SemaphoreType.DMA,
            pltpu.SemaphoreType.DMA,
            pltpu.SemaphoreType.DMA,
            pltpu.SemaphoreType.DMA,
            pltpu.SemaphoreType.DMA,
        ],
    )
    def spmm(x_hbm, src_hbm, dst_hbm, val_hbm, out_hbm, srcA, dstA, valA,
             rows, hacc, isem, gsem0, gsem1, gsem2, ssem0, ssem1, ssem2):
        cid = lax.axis_index("c")
        sid = lax.axis_index("s")
        wid = cid * NS + sid
        gsem = (gsem0, gsem1, gsem2)
        ssem = (ssem0, ssem1, ssem2)
        zeros = jnp.zeros((L,), jnp.float32)

        def issue_idx(blk, bslot):
            pltpu.async_copy(src_hbm.at[wid, blk], srcA.at[bslot], isem)
            pltpu.async_copy(dst_hbm.at[wid, blk], dstA.at[bslot], isem)
            pltpu.async_copy(val_hbm.at[wid, blk], valA.at[bslot], isem)

        def wait_idx(bslot):
            pltpu.make_async_copy(src_hbm.at[wid, 0], srcA.at[bslot], isem).wait()
            pltpu.make_async_copy(src_hbm.at[wid, 0], dstA.at[bslot], isem).wait()
            pltpu.make_async_copy(val_hbm.at[wid, 0], valA.at[bslot], isem).wait()

        def wait_gather(b):
            pltpu.make_async_copy(
                x_hbm.at[srcA.at[0, 0]], rows.at[b], gsem[b]).wait()

        def wait_scatter(b):
            pltpu.make_async_copy(
                rows.at[b], hacc.at[dstA.at[0, 0]], ssem[b]).wait()

        issue_idx(0, 0)

        def zrow(i, carry):
            for j in range(D // L):
                rows[0, i, pl.ds(L * j, L)] = zeros
            return carry
        lax.fori_loop(0, C, zrow, 0)
        for k in range(rpw // C):
            pltpu.sync_copy(rows.at[0],
                            hacc.at[pl.ds(sid * rpw + k * C, C)])
        if rpw % C:
            pltpu.sync_copy(rows.at[0, pl.ds(0, rpw % C)],
                            hacc.at[pl.ds(sid * rpw + (rpw // C) * C, rpw % C)])
        plsc.subcore_barrier()

        wait_idx(0)
        pltpu.async_copy(x_hbm.at[srcA.at[0, 0]], rows.at[0], gsem0)

        def blk_body(blk, carry):
            bp = blk % 2
            bq = (blk + 1) % 2
            for j in range(BK):
                b, bn = j % 3, (j + 1) % 3
                wait_gather(b)
                if j >= 2:
                    wait_scatter(bn)
                else:
                    @pl.when(blk > 0)
                    def _():
                        wait_scatter(bn)
                if j == 1:
                    @pl.when(blk + 1 < nblk)
                    def _():
                        issue_idx(blk + 1, bq)
                if j < BK - 1:
                    pltpu.async_copy(
                        x_hbm.at[srcA.at[bp, j + 1]], rows.at[bn], gsem[bn])
                else:
                    @pl.when(blk + 1 < nblk)
                    def _():
                        wait_idx(bq)
                        pltpu.async_copy(
                            x_hbm.at[srcA.at[bq, 0]], rows.at[bn], gsem[bn])

                def scale_grp(g, c2):
                    vv = valA[bp, j, pl.ds(g * L, L)]
                    for ri in range(L):
                        v = jnp.full((L,), vv[ri])
                        r = g * L + ri
                        for jj in range(D // L):
                            rows[b, r, pl.ds(L * jj, L)] = (
                                rows[b, r, pl.ds(L * jj, L)] * v)
                    return c2
                lax.fori_loop(0, C // L, scale_grp, 0)
                pltpu.async_copy(
                    rows.at[b], hacc.at[dstA.at[bp, j]], ssem[b], add=True)
            return carry
        lax.fori_loop(0, nblk, blk_body, 0)
        wait_scatter((BK * nblk - 2) % 3)
        wait_scatter((BK * nblk - 1) % 3)
        plsc.subcore_barrier()

        @pl.when(sid == 0)
        def _():
            pltpu.sync_copy(hacc, out_hbm.at[cid])

    return spmm


def _tc_epilogue(hp, h0, W):
    N, D = h0.shape
    R = 2000
    assert N % R == 0

    def body(hp_ref, h0_ref, w_ref, out_ref):
        h = (hp_ref[0] + hp_ref[1]) * (1.0 - ALPHA)
        support = h + ALPHA * h0_ref[...]
        out_ref[...] = (
            BETA * jnp.dot(support, w_ref[...],
                           preferred_element_type=jnp.float32)
            + (1.0 - BETA) * support)

    return pl.pallas_call(
        body,
        grid=(N // R,),
        in_specs=[
            pl.BlockSpec((NC, R, D), lambda i: (0, i, 0)),
            pl.BlockSpec((R, D), lambda i: (i, 0)),
            pl.BlockSpec((D, D), lambda i: (0, 0)),
        ],
        out_specs=pl.BlockSpec((R, D), lambda i: (i, 0)),
        out_shape=jax.ShapeDtypeStruct((N, D), jnp.float32),
    )(hp, h0, W)


def kernel(input, adj_edge_index, adj_values, h0, W, lth):
    N, D = input.shape
    E = adj_values.shape[0]
    nblk = -(-E // (NW * C * BK))
    e_pad = NW * nblk * BK * C - E
    pad_idx = jnp.arange(e_pad, dtype=jnp.int32) % C
    src = jnp.concatenate([adj_edge_index[0], pad_idx])
    dst = jnp.concatenate([adj_edge_index[1], pad_idx])
    vals = jnp.concatenate([adj_values, jnp.zeros((e_pad,), jnp.float32)])
    src = src.reshape(NW, nblk, BK, C)
    dst = dst.reshape(NW, nblk, BK, C)
    vals = vals.reshape(NW, nblk, BK, C)
    hp = _sc_spmm_kernel(N, D, nblk)(input, src, dst, vals)
    return _tc_epilogue(hp, h0, W)


# speedup vs baseline: 1.1596x; 1.1596x over previous
import functools
import math

import jax
import jax.numpy as jnp
from jax import lax
from jax.experimental import pallas as pl
from jax.experimental.pallas import tpu as pltpu
from jax.experimental.pallas import tpu_sc as plsc

ALPHA = 0.1
THETA = 0.5
BETA = math.log(THETA / 2 + 1.0)

NC = 2
NS = 16
NW = NC * NS
L = 16
C = 96
BK = 9


def _sc_spmm_kernel(N, D, nblk):
    rpw = N // NS

    mesh = plsc.VectorSubcoreMesh(core_axis_name="c", subcore_axis_name="s")

    @functools.partial(
        pl.kernel,
        out_type=jax.ShapeDtypeStruct((NC, N, D), jnp.float32),
        mesh=mesh,
        scratch_types=[
            pltpu.VMEM((2, BK, C), jnp.int32),
            pltpu.VMEM((2, BK, C), jnp.int32),
            pltpu.VMEM((2, BK, C), jnp.float32),
            pltpu.VMEM((3, C, D), jnp.float32),
            pltpu.VMEM_SHARED((N, D), jnp.float32),
            pltpu.SemaphoreType.DMA,
            pltpu.SemaphoreType.DMA,
            pltpu.SemaphoreType.DMA,
            pltpu.SemaphoreType.DMA,
        ],
    )
    def spmm(x_hbm, src_hbm, dst_hbm, val_hbm, out_hbm,
             srcA, dstA, valA, rows, hacc, isem, gsem0, gsem1, gsem2):
        cid = lax.axis_index("c")
        sid = lax.axis_index("s")
        wid = cid * NS + sid
        gsem = (gsem0, gsem1, gsem2)
        zeros = jnp.zeros((L,), jnp.float32)

        def issue_idx(blk, bslot):
            pltpu.async_copy(src_hbm.at[wid, blk], srcA.at[bslot], isem)
            pltpu.async_copy(dst_hbm.at[wid, blk], dstA.at[bslot], isem)
            pltpu.async_copy(val_hbm.at[wid, blk], valA.at[bslot], isem)

        def wait_idx(bslot):
            pltpu.make_async_copy(src_hbm.at[wid, 0], srcA.at[bslot], isem).wait()
            pltpu.make_async_copy(src_hbm.at[wid, 0], dstA.at[bslot], isem).wait()
            pltpu.make_async_copy(val_hbm.at[wid, 0], valA.at[bslot], isem).wait()

        def wait_gather(p, bslot, j):
            pltpu.make_async_copy(
                x_hbm.at[srcA.at[bslot, j]], rows.at[p], gsem[p]).wait()

        issue_idx(0, 0)

        def zrow(i, carry):
            for j in range(D // L):
                rows[0, i, pl.ds(L * j, L)] = zeros
            return carry
        lax.fori_loop(0, C, zrow, 0)
        for k in range(rpw // C):
            pltpu.sync_copy(rows.at[0], hacc.at[pl.ds(sid * rpw + k * C, C)])
        if rpw % C:
            pltpu.sync_copy(rows.at[0, pl.ds(0, rpw % C)],
                            hacc.at[pl.ds(sid * rpw + (rpw // C) * C, rpw % C)])
        plsc.subcore_barrier()

        wait_idx(0)
        pltpu.async_copy(x_hbm.at[srcA.at[0, 0]], rows.at[0], gsem0)
        pltpu.async_copy(x_hbm.at[srcA.at[0, 1]], rows.at[1], gsem1)

        def blk_body(blk, carry):
            bp = blk % 2
            bq = (blk + 1) % 2
            for j in range(BK):
                p, q = j % 3, (j + 2) % 3
                wait_gather(p, bp, j)
                if j == 0:
                    @pl.when(blk + 1 < nblk)
                    def _():
                        issue_idx(blk + 1, bq)
                if j < BK - 2:
                    pltpu.async_copy(
                        x_hbm.at[srcA.at[bp, j + 2]], rows.at[q], gsem[q])
                elif j == BK - 2:
                    @pl.when(blk + 1 < nblk)
                    def _():
                        wait_idx(bq)
                        pltpu.async_copy(
                            x_hbm.at[srcA.at[bq, 0]], rows.at[q], gsem[q])
                else:
                    @pl.when(blk + 1 < nblk)
                    def _():
                        pltpu.async_copy(
                            x_hbm.at[srcA.at[bq, 1]], rows.at[q], gsem[q])

                def scale_grp(g, c2):
                    vv = valA[bp, j, pl.ds(g * L, L)]
                    for ri in range(L):
                        v = jnp.full((L,), vv[ri])
                        r = g * L + ri
                        for jj in range(D // L):
                            rows[p, r, pl.ds(L * jj, L)] = (
                                rows[p, r, pl.ds(L * jj, L)] * v)
                    return c2
                lax.fori_loop(0, C // L, scale_grp, 0)
                pltpu.sync_copy(rows.at[p], hacc.at[dstA.at[bp, j]], add=True)
            return carry
        lax.fori_loop(0, nblk, blk_body, 0)
        plsc.subcore_barrier()

        @pl.when(sid == 0)
        def _():
            pltpu.sync_copy(hacc, out_hbm.at[cid])

    return spmm


def _tc_epilogue(hp, h0, W):
    N, D = h0.shape
    R = 2000
    assert N % R == 0

    def body(hp_ref, h0_ref, w_ref, out_ref):
        h = (hp_ref[0] + hp_ref[1]) * (1.0 - ALPHA)
        support = h + ALPHA * h0_ref[...]
        out_ref[...] = (
            BETA * jnp.dot(support, w_ref[...],
                           preferred_element_type=jnp.float32)
            + (1.0 - BETA) * support)

    return pl.pallas_call(
        body,
        grid=(N // R,),
        in_specs=[
            pl.BlockSpec((NC, R, D), lambda i: (0, i, 0)),
            pl.BlockSpec((R, D), lambda i: (i, 0)),
            pl.BlockSpec((D, D), lambda i: (0, 0)),
        ],
        out_specs=pl.BlockSpec((R, D), lambda i: (i, 0)),
        out_shape=jax.ShapeDtypeStruct((N, D), jnp.float32),
    )(hp, h0, W)


def kernel(input, adj_edge_index, adj_values, h0, W, lth):
    N, D = input.shape
    E = adj_values.shape[0]
    nblk = -(-E // (NW * C * BK))
    e_pad = NW * nblk * BK * C - E
    pad_idx = jnp.arange(e_pad, dtype=jnp.int32) % C
    src = jnp.concatenate([adj_edge_index[0], pad_idx])
    dst = jnp.concatenate([adj_edge_index[1], pad_idx])
    vals = jnp.concatenate([adj_values, jnp.zeros((e_pad,), jnp.float32)])
    src = src.reshape(NW, nblk, BK, C)
    dst = dst.reshape(NW, nblk, BK, C)
    vals = vals.reshape(NW, nblk, BK, C)
    hp = _sc_spmm_kernel(N, D, nblk)(input, src, dst, vals)
    return _tc_epilogue(hp, h0, W)
